# Initial kernel scaffold; baseline (speedup 1.0000x reference)
import jax
import jax.numpy as jnp
from jax.experimental import pallas as pl


def kernel(combined_embeddings, params, gene_node_indices, dna_node_indices, edge_index, edge_attr):
    raise NotImplementedError("write your pallas kernel here")


# speedup vs baseline: 25.9152x; 25.9152x over previous
import functools
import math

import jax
import jax.numpy as jnp
from jax import lax
from jax.experimental import pallas as pl
from jax.experimental.pallas import tpu as pltpu
from jax.experimental.pallas import tpu_sc as plsc

HID = 272
NPAD = 10240
CHUNK = 128
NC, NS = 2, 16
NW = NC * NS


def _sc_gather(table, idx):
  B = idx.shape[0]
  D = table.shape[1]
  nchunks = B // CHUNK
  iters = (nchunks + NW - 1) // NW
  mesh = plsc.VectorSubcoreMesh(core_axis_name="c", subcore_axis_name="s")

  @functools.partial(
      pl.kernel, mesh=mesh,
      out_type=jax.ShapeDtypeStruct((B, D), jnp.float32),
      scratch_types=[
          pltpu.VMEM((CHUNK,), jnp.int32),
          pltpu.VMEM((CHUNK, D), jnp.float32),
          pltpu.SemaphoreType.DMA,
      ],
  )
  def k(table_hbm, idx_hbm, out_hbm, idx_v, rows_v, sem):
    wid = lax.axis_index("s") * NC + lax.axis_index("c")

    def body(i, carry):
      cid = wid + i * NW

      @pl.when(cid < nchunks)
      def _():
        base = cid * CHUNK
        pltpu.sync_copy(idx_hbm.at[pl.ds(base, CHUNK)], idx_v)
        pltpu.async_copy(table_hbm.at[idx_v], rows_v, sem).wait()
        pltpu.sync_copy(rows_v, out_hbm.at[pl.ds(base, CHUNK)])

      return carry

    lax.fori_loop(0, iters, body, 0)

  return k(table, idx)


def _sc_scatter_add(rows, idx, zeros, nsl):
  E = rows.shape[0]
  nchunks = E // CHUNK
  iters = (nchunks + NW - 1) // NW
  rpn = NPAD // NS
  mesh = plsc.VectorSubcoreMesh(core_axis_name="c", subcore_axis_name="s")

  @functools.partial(
      pl.kernel, mesh=mesh,
      out_type=jax.ShapeDtypeStruct((NC, nsl, NPAD, 128), jnp.float32),
      scratch_types=[
          pltpu.VMEM((CHUNK,), jnp.int32),
          pltpu.VMEM((CHUNK, 128), jnp.float32),
          pltpu.VMEM_SHARED((NPAD, 128), jnp.float32),
          pltpu.SemaphoreType.DMA,
      ],
  )
  def k(rows_hbm, idx_hbm, zeros_hbm, out_hbm, idx_v, rows_v, acc, sem):
    c = lax.axis_index("c")
    s = lax.axis_index("s")
    wid = s * NC + c
    for sl in range(nsl):
      pltpu.sync_copy(zeros_hbm.at[pl.ds(s * rpn, rpn)],
                      acc.at[pl.ds(s * rpn, rpn)])
      plsc.subcore_barrier()

      def body(i, carry):
        cid = wid + i * NW

        @pl.when(cid < nchunks)
        def _():
          base = cid * CHUNK
          pltpu.sync_copy(idx_hbm.at[pl.ds(base, CHUNK)], idx_v)
          pltpu.sync_copy(rows_hbm.at[pl.ds(base, CHUNK),
                                      pl.ds(sl * 128, 128)], rows_v)
          pltpu.sync_copy(rows_v, acc.at[idx_v], add=True)

        return carry

      lax.fori_loop(0, iters, body, 0)
      plsc.subcore_barrier()
      pltpu.sync_copy(acc.at[pl.ds(s * rpn, rpn)],
                      out_hbm.at[c, sl].at[pl.ds(s * rpn, rpn)])
      plsc.subcore_barrier()

  return k(rows, idx, zeros)


def _mm_kernel(x_ref, w_ref, b_ref, o_ref):
  o_ref[...] = jnp.dot(x_ref[...], w_ref[...],
                       preferred_element_type=jnp.float32) + b_ref[...]


def _tc_matmul_bias(x, w, b, bn=400):
  n, kdim = x.shape
  m = w.shape[1]
  return pl.pallas_call(
      _mm_kernel,
      grid=(n // bn,),
      in_specs=[
          pl.BlockSpec((bn, kdim), lambda i: (i, 0)),
          pl.BlockSpec((kdim, m), lambda i: (0, 0)),
          pl.BlockSpec((1, m), lambda i: (0, 0)),
      ],
      out_specs=pl.BlockSpec((bn, m), lambda i: (i, 0)),
      out_shape=jax.ShapeDtypeStruct((n, m), jnp.float32),
  )(x, w, b.reshape(1, m))


def _node_prep_kernel(yd_ref, g_ref, bq_ref, bk_ref, bv_ref, wo_ref, bo_ref,
                      pool_ref, wq_ref, bq1_ref, wk_ref, bk1_ref,
                      wv_ref, bv1_ref, ws_ref, bs1_ref,
                      xin_ref, q_ref, k_ref, v_ref, sk_ref):
  dc0 = yd_ref[...]
  qf = jnp.dot(dc0, bq_ref[...], preferred_element_type=jnp.float32)
  kf = jnp.dot(dc0, bk_ref[...], preferred_element_type=jnp.float32)
  vf = jnp.dot(dc0, bv_ref[...], preferred_element_type=jnp.float32)
  s = jnp.dot(qf * kf, pool_ref[...],
              preferred_element_type=jnp.float32) / 12.0
  s = s - jnp.max(s, axis=-1, keepdims=True)
  es = jnp.exp(s)
  p = es / jnp.sum(es, axis=-1, keepdims=True)
  vw = vf * jnp.dot(p, pool_ref[...].T, preferred_element_type=jnp.float32)
  dc = jnp.dot(vw, wo_ref[...], preferred_element_type=jnp.float32) + bo_ref[...]
  g = g_ref[...]
  g = g / jnp.maximum(jnp.sqrt(jnp.sum(g * g, axis=-1, keepdims=True)), 1e-12)
  xin = jnp.concatenate([dc, g], axis=1)
  xin_ref[...] = xin
  q_ref[...] = jnp.dot(xin, wq_ref[...],
                       preferred_element_type=jnp.float32) + bq1_ref[...]
  k_ref[...] = jnp.dot(xin, wk_ref[...],
                       preferred_element_type=jnp.float32) + bk1_ref[...]
  v_ref[...] = jnp.dot(xin, wv_ref[...],
                       preferred_element_type=jnp.float32) + bv1_ref[...]
  sk_ref[...] = jnp.dot(xin, ws_ref[...],
                        preferred_element_type=jnp.float32) + bs1_ref[...]


def _tc_node_prep(yd, gemb, bq, bk, bv, wo, bo, pool,
                  wq, bq1, wk, bk1, wv, bv1, ws, bs1, bn=400):
  n = yd.shape[0]
  dp = wq.shape[1]
  full = lambda a: pl.BlockSpec(a.shape, lambda i: tuple(0 for _ in a.shape))
  row = lambda a: pl.BlockSpec((1, a.shape[0]), lambda i: (0, 0))
  return pl.pallas_call(
      _node_prep_kernel,
      grid=(n // bn,),
      in_specs=[
          pl.BlockSpec((bn, 256), lambda i: (i, 0)),
          pl.BlockSpec((bn, 128), lambda i: (i, 0)),
          full(bq), full(bk), full(bv), full(wo), row(bo),
          full(pool), full(wq), row(bq1), full(wk), row(bk1),
          full(wv), row(bv1), full(ws), row(bs1),
      ],
      out_specs=[
          pl.BlockSpec((bn, 272), lambda i: (i, 0)),
          pl.BlockSpec((bn, dp), lambda i: (i, 0)),
          pl.BlockSpec((bn, dp), lambda i: (i, 0)),
          pl.BlockSpec((bn, dp), lambda i: (i, 0)),
          pl.BlockSpec((bn, 272), lambda i: (i, 0)),
      ],
      out_shape=[
          jax.ShapeDtypeStruct((n, 272), jnp.float32),
          jax.ShapeDtypeStruct((n, dp), jnp.float32),
          jax.ShapeDtypeStruct((n, dp), jnp.float32),
          jax.ShapeDtypeStruct((n, dp), jnp.float32),
          jax.ShapeDtypeStruct((n, 272), jnp.float32),
      ],
  )(yd, gemb, bq, bk, bv, wo, bo.reshape(1, -1), pool,
    wq, bq1.reshape(1, -1), wk, bk1.reshape(1, -1),
    wv, bv1.reshape(1, -1), ws, bs1.reshape(1, -1))


def _edge_logits_kernel(qg_ref, kg_ref, oh_ref, ew_ref, hpool_ref,
                        l_ref, bmax_ref):
  eg = jnp.dot(oh_ref[...], ew_ref[...], preferred_element_type=jnp.float32)
  prod = qg_ref[...] * (kg_ref[...] + eg)
  l = jnp.dot(prod, hpool_ref[...],
              preferred_element_type=jnp.float32) / math.sqrt(float(HID))
  l_ref[...] = l
  bmax_ref[...] = jnp.max(l, axis=0, keepdims=True)[None]


def _tc_edge_logits(qg, kg, onehot, ew, hpool, be=2000):
  e, d = qg.shape
  h = hpool.shape[1]
  nb = e // be
  return pl.pallas_call(
      _edge_logits_kernel,
      grid=(nb,),
      in_specs=[
          pl.BlockSpec((be, d), lambda i: (i, 0)),
          pl.BlockSpec((be, d), lambda i: (i, 0)),
          pl.BlockSpec((be, 8), lambda i: (i, 0)),
          pl.BlockSpec((8, d), lambda i: (0, 0)),
          pl.BlockSpec((d, h), lambda i: (0, 0)),
      ],
      out_specs=[
          pl.BlockSpec((be, h), lambda i: (i, 0)),
          pl.BlockSpec((1, 1, h), lambda i: (i, 0, 0)),
      ],
      out_shape=[
          jax.ShapeDtypeStruct((e, h), jnp.float32),
          jax.ShapeDtypeStruct((nb, 1, h), jnp.float32),
      ],
  )(qg, kg, onehot, ew, hpool)


def _edge_weights_kernel(l_ref, gmax_ref, vg_ref, oh_ref, ew_ref, hexp_ref,
                         sel_ref, ext_ref):
  ex = jnp.exp(l_ref[...] - gmax_ref[...])
  eg = jnp.dot(oh_ref[...], ew_ref[...], preferred_element_type=jnp.float32)
  mult = jnp.dot(ex, hexp_ref[...], preferred_element_type=jnp.float32)
  ext_ref[...] = (vg_ref[...] + eg) * mult + jnp.dot(
      ex, sel_ref[...], preferred_element_type=jnp.float32)


def _tc_edge_weights(l, gmax, vg, onehot, ew, hexp, sel, be=2000):
  e, d = vg.shape
  h = l.shape[1]
  nb = e // be
  return pl.pallas_call(
      _edge_weights_kernel,
      grid=(nb,),
      in_specs=[
          pl.BlockSpec((be, h), lambda i: (i, 0)),
          pl.BlockSpec((1, h), lambda i: (0, 0)),
          pl.BlockSpec((be, d), lambda i: (i, 0)),
          pl.BlockSpec((be, 8), lambda i: (i, 0)),
          pl.BlockSpec((8, d), lambda i: (0, 0)),
          pl.BlockSpec((h, d), lambda i: (0, 0)),
          pl.BlockSpec((h, d), lambda i: (0, 0)),
      ],
      out_specs=pl.BlockSpec((be, d), lambda i: (i, 0)),
      out_shape=jax.ShapeDtypeStruct((e, d), jnp.float32),
  )(l, gmax, vg, onehot, ew, hexp, sel)


def _post_kernel(h, cat, xin_ref, num_ref, den_ref, sk_ref,
                 lng_ref, lnb_ref, wgate_ref, bgate_ref,
                 wq_ref, bq_ref, wk_ref, bk_ref, wv_ref, bv_ref,
                 ws_ref, bs_ref,
                 x_ref, q_ref, k_ref, v_ref, sk2_ref):
  num = num_ref[0] + num_ref[1]
  den = den_ref[0] + den_ref[1]
  xin = xin_ref[...]
  parts = []
  for hh in range(h):
    d = den[:, hh:hh + 1]
    inv = jnp.where(d > 0, 1.0 / jnp.where(d > 0, d, 1.0), 0.0)
    parts.append(num[:, hh, :] * inv)
  if cat:
    o = jnp.concatenate(parts, axis=1)
  else:
    o = parts[0]
    for pp in parts[1:]:
      o = o + pp
    o = o / float(h)
  x1 = o + sk_ref[...]
  mu = jnp.mean(x1, axis=-1, keepdims=True)
  var = jnp.mean((x1 - mu) ** 2, axis=-1, keepdims=True)
  x1 = (x1 - mu) * lax.rsqrt(var + 1e-5) * lng_ref[...] + lnb_ref[...]
  x1 = 0.5 * x1 * (1.0 + lax.erf(x1 / math.sqrt(2.0)))
  z = jnp.concatenate([xin, x1], axis=1)
  gate = jnp.dot(z, wgate_ref[...], preferred_element_type=jnp.float32)
  gate = jax.nn.sigmoid(gate + bgate_ref[...])[:, 0:1]
  x = gate * xin + (1.0 - gate) * x1
  x_ref[...] = x
  q_ref[...] = jnp.dot(x, wq_ref[...],
                       preferred_element_type=jnp.float32) + bq_ref[...]
  k_ref[...] = jnp.dot(x, wk_ref[...],
                       preferred_element_type=jnp.float32) + bk_ref[...]
  v_ref[...] = jnp.dot(x, wv_ref[...],
                       preferred_element_type=jnp.float32) + bv_ref[...]
  sk2_ref[...] = jnp.dot(x, ws_ref[...],
                         preferred_element_type=jnp.float32) + bs_ref[...]


def _tc_post(h, cat, xin, num, den, sk, lng, lnb, wgate, bgate,
             wq, bq, wk, bk, wv, bv, ws, bs, bn=400):
  n = xin.shape[0]
  dp = wq.shape[1]
  sp = ws.shape[1]
  gw = wgate.shape[1]
  full = lambda a: pl.BlockSpec(a.shape, lambda i: tuple(0 for _ in a.shape))
  row = lambda a: pl.BlockSpec((1, a.shape[0]), lambda i: (0, 0))
  return pl.pallas_call(
      functools.partial(_post_kernel, h, cat),
      grid=(n // bn,),
      in_specs=[
          pl.BlockSpec((bn, 272), lambda i: (i, 0)),
          pl.BlockSpec((2, bn, h, 272), lambda i: (0, i, 0, 0)),
          pl.BlockSpec((2, bn, 8), lambda i: (0, i, 0)),
          pl.BlockSpec((bn, 272), lambda i: (i, 0)),
          row(lng), row(lnb),
          full(wgate), row(bgate),
          full(wq), row(bq), full(wk), row(bk), full(wv), row(bv),
          full(ws), row(bs),
      ],
      out_specs=[
          pl.BlockSpec((bn, 272), lambda i: (i, 0)),
          pl.BlockSpec((bn, dp), lambda i: (i, 0)),
          pl.BlockSpec((bn, dp), lambda i: (i, 0)),
          pl.BlockSpec((bn, dp), lambda i: (i, 0)),
          pl.BlockSpec((bn, sp), lambda i: (i, 0)),
      ],
      out_shape=[
          jax.ShapeDtypeStruct((n, 272), jnp.float32),
          jax.ShapeDtypeStruct((n, dp), jnp.float32),
          jax.ShapeDtypeStruct((n, dp), jnp.float32),
          jax.ShapeDtypeStruct((n, dp), jnp.float32),
          jax.ShapeDtypeStruct((n, sp), jnp.float32),
      ],
  )(xin, num, den, sk, lng.reshape(1, -1), lnb.reshape(1, -1),
    wgate, bgate.reshape(1, -1), wq, bq.reshape(1, -1),
    wk, bk.reshape(1, -1), wv, bv.reshape(1, -1), ws, bs.reshape(1, -1))


def _padc(a, cols):
  return jnp.pad(a, ((0, 0), (0, cols - a.shape[1])))


def kernel(combined_embeddings, params, gene_node_indices, dna_node_indices,
           edge_index, edge_attr):
  pdict = params
  n = combined_embeddings.shape[0]

  wc = jnp.zeros((4096, 256), jnp.float32)
  wc = wc.at[0:768, 0:64].set(pdict['W1'].T)
  wc = wc.at[768:1536, 64:128].set(pdict['W2'].T)
  wc = wc.at[1536:4096, 128:144].set(pdict['W3'].T)
  bc = jnp.pad(jnp.concatenate([pdict['b1'], pdict['b2'], pdict['b3']]),
               (0, 112))
  eye4 = jnp.eye(4, dtype=jnp.float32)
  pad_rows = lambda a: jnp.pad(a, ((0, 256 - a.shape[0]), (0, 0)))
  bq = pad_rows(jnp.kron(eye4, pdict['mha_Wq'].T))
  bk = pad_rows(jnp.kron(eye4, pdict['mha_Wk'].T))
  bv = pad_rows(jnp.kron(eye4, pdict['mha_Wv'].T))
  pool4 = jnp.kron(eye4, jnp.ones((36, 1), jnp.float32))
  wo = pdict['mha_Wo'].T

  c1, c2 = pdict['conv1'], pdict['conv2']
  dp1, dp2 = 640, 384
  wq1 = _padc(c1['Wq'].T, dp1)
  wk1 = _padc(c1['Wk'].T, dp1)
  wv1 = _padc(c1['Wv'].T, dp1)
  bq1 = jnp.pad(c1['bq'], (0, dp1 - 544))
  bk1 = jnp.pad(c1['bk'], (0, dp1 - 544))
  bv1 = jnp.pad(c1['bv'], (0, dp1 - 544))
  ws1 = c1['Wskip'].T
  bs1 = c1['bskip']
  wq2 = _padc(c2['Wq'].T, dp2)
  wk2 = _padc(c2['Wk'].T, dp2)
  wv2 = _padc(c2['Wv'].T, dp2)
  bq2 = jnp.pad(c2['bq'], (0, dp2 - 272))
  bk2 = jnp.pad(c2['bk'], (0, dp2 - 272))
  bv2 = jnp.pad(c2['bv'], (0, dp2 - 272))
  ws2 = c2['Wskip'].T
  bs2 = c2['bskip']
  ew1 = _padc(pdict['pathway_emb'] @ c1['We'].T, dp1)
  ew2 = _padc(pdict['pathway_emb'] @ c2['We'].T, dp2)

  didx = dna_node_indices.astype(jnp.int32)
  gidx = jnp.clip(gene_node_indices, 0,
                  pdict['gene_emb'].shape[0] - 1).astype(jnp.int32)
  src = edge_index[0].astype(jnp.int32)
  dst = edge_index[1].astype(jnp.int32)
  pid = edge_attr[:, 0]
  pid = jnp.where(pid < 0, pdict['pathway_emb'].shape[0] - 1, pid)
  onehot = (pid[:, None] == jnp.arange(8)[None, :]).astype(jnp.float32)

  pad = NPAD - n
  didx_p = jnp.pad(didx, (0, pad))
  gidx_p = jnp.pad(gidx, (0, pad))
  zeros128 = jnp.zeros((NPAD, 128), jnp.float32)

  y = _tc_matmul_bias(combined_embeddings, wc, bc)
  yd = _sc_gather(y, didx_p)[:n]
  gemb = _sc_gather(pdict['gene_emb'], gidx_p)[:n]

  xin, q1, k1, v1, sk1 = _tc_node_prep(
      yd, gemb, bq, bk, bv, wo, pdict['mha_bo'], pool4,
      wq1, bq1, wk1, bk1, wv1, bv1, ws1, bs1)

  def edge_stage(q, k, v, ew, h):
    d = h * HID
    dpp = q.shape[1]
    nsl = dpp // 128
    hp = jnp.zeros((dpp, h), jnp.float32)
    for i in range(h):
      hp = hp.at[i * HID:(i + 1) * HID, i].set(1.0)
    sel = jnp.zeros((h, dpp), jnp.float32)
    for i in range(h):
      sel = sel.at[i, d + i].set(1.0)
    qg = _sc_gather(q, dst)
    kg = _sc_gather(k, src)
    vg = _sc_gather(v, src)
    l, bmax = _tc_edge_logits(qg, kg, onehot, ew, hp)
    gmax = jnp.max(bmax.reshape(-1, h), axis=0, keepdims=True)
    ext = _tc_edge_weights(l, gmax, vg, onehot, ew, hp.T, sel)
    parts = _sc_scatter_add(ext, dst, zeros128, nsl)
    full = parts.transpose(0, 2, 1, 3).reshape(2, NPAD, dpp)[:, :n]
    num = full[:, :, :d].reshape(2, n, h, HID)
    den = jnp.pad(full[:, :, d:d + h], ((0, 0), (0, 0), (0, 8 - h)))
    return num, den

  num1, den1 = edge_stage(q1, k1, v1, ew1, 2)
  gate1w = jnp.pad(pdict['bl1_W'].T, ((0, 0), (0, 7)))
  gate1b = jnp.pad(pdict['bl1_b'], (0, 7))
  x, q2, k2, v2, sk2 = _tc_post(
      2, False, xin, num1, den1, sk1, pdict['ln1_g'], pdict['ln1_b'],
      gate1w, gate1b, wq2, bq2, wk2, bk2, wv2, bv2, ws2, bs2)

  num2, den2 = edge_stage(q2, k2, v2, ew2, 1)
  gate2w = jnp.pad(pdict['bl2_W'].T, ((0, 0), (0, 7)))
  gate2b = jnp.pad(pdict['bl2_b'], (0, 7))
  zz = jnp.zeros((272, 128), jnp.float32)
  zb = jnp.zeros((128,), jnp.float32)
  out, _, _, _, _ = _tc_post(
      1, True, x, num2, den2, sk2, pdict['ln2_g'], pdict['ln2_b'],
      gate2w, gate2b, zz, zb, zz, zb, zz, zb, zz, zb)
  return out, pdict['pathway_emb']


# speedup vs baseline: 29.9388x; 1.1553x over previous
import functools
import math

import jax
import jax.numpy as jnp
from jax import lax
from jax.experimental import pallas as pl
from jax.experimental.pallas import tpu as pltpu
from jax.experimental.pallas import tpu_sc as plsc

HID = 272
NPAD = 10240
CHUNK = 128
NC, NS = 2, 16
NW = NC * NS


def _gchunk(d):
  for c in (128, 64, 32, 16, 8):
    if 2 * c * d * 4 <= 460_000:
      return c
  return 8


def _sc_gather(table, idx):
  B = idx.shape[0]
  D = table.shape[1]
  chunk = _gchunk(D)
  assert B % chunk == 0
  nchunks = B // chunk
  iters = (nchunks + NW - 1) // NW
  iters2 = (iters + 1) // 2
  mesh = plsc.VectorSubcoreMesh(core_axis_name="c", subcore_axis_name="s")

  @functools.partial(
      pl.kernel, mesh=mesh,
      out_type=jax.ShapeDtypeStruct((B, D), jnp.float32),
      scratch_types=[
          pltpu.VMEM((chunk,), jnp.int32),
          pltpu.VMEM((chunk,), jnp.int32),
          pltpu.VMEM((chunk, D), jnp.float32),
          pltpu.VMEM((chunk, D), jnp.float32),
          pltpu.SemaphoreType.DMA,
          pltpu.SemaphoreType.DMA,
      ],
  )
  def k(table_hbm, idx_hbm, out_hbm, i0, i1, r0, r1, s0, s1):
    wid = lax.axis_index("s") * NC + lax.axis_index("c")
    idx_v = (i0, i1)
    rows_v = (r0, r1)
    sems = (s0, s1)

    def start(i, b):
      @pl.when(i * NW + wid < nchunks)
      def _():
        base = (i * NW + wid) * chunk
        pltpu.sync_copy(idx_hbm.at[pl.ds(base, chunk)], idx_v[b])
        pltpu.async_copy(table_hbm.at[idx_v[b]], rows_v[b], sems[b])

    def finish(i, b):
      @pl.when(i * NW + wid < nchunks)
      def _():
        base = (i * NW + wid) * chunk
        pltpu.make_async_copy(table_hbm.at[idx_v[b]], rows_v[b],
                              sems[b]).wait()
        pltpu.sync_copy(rows_v[b], out_hbm.at[pl.ds(base, chunk)])

    start(0, 0)

    def body(i2, carry):
      i = i2 * 2
      start(i + 1, 1)
      finish(i, 0)
      start(i + 2, 0)
      finish(i + 1, 1)
      return carry

    lax.fori_loop(0, iters2, body, 0)

  return k(table, idx)


def _sc_scatter_add(rows, idx, zeros, nsl):
  E = rows.shape[0]
  nchunks = E // CHUNK
  iters = (nchunks + NW - 1) // NW
  rpn = NPAD // NS
  mesh = plsc.VectorSubcoreMesh(core_axis_name="c", subcore_axis_name="s")

  @functools.partial(
      pl.kernel, mesh=mesh,
      out_type=jax.ShapeDtypeStruct((NC, nsl, NPAD, 128), jnp.float32),
      scratch_types=[
          pltpu.VMEM((CHUNK,), jnp.int32),
          pltpu.VMEM((CHUNK,), jnp.int32),
          pltpu.VMEM((CHUNK, 128), jnp.float32),
          pltpu.VMEM((CHUNK, 128), jnp.float32),
          pltpu.VMEM_SHARED((NPAD, 128), jnp.float32),
          pltpu.SemaphoreType.DMA,
          pltpu.SemaphoreType.DMA,
      ],
  )
  def k(rows_hbm, idx_hbm, zeros_hbm, out_hbm, i0, i1, r0, r1, acc,
        s0, s1):
    c = lax.axis_index("c")
    s = lax.axis_index("s")
    wid = s * NC + c
    idx_v = (i0, i1)
    rows_v = (r0, r1)
    sems = (s0, s1)
    for sl in range(nsl):
      pltpu.sync_copy(zeros_hbm.at[pl.ds(s * rpn, rpn)],
                      acc.at[pl.ds(s * rpn, rpn)])
      plsc.subcore_barrier()

      def start(i, b):
        @pl.when(i * NW + wid < nchunks)
        def _():
          base = (i * NW + wid) * CHUNK
          pltpu.async_copy(idx_hbm.at[pl.ds(base, CHUNK)], idx_v[b], sems[b])
          pltpu.async_copy(rows_hbm.at[pl.ds(base, CHUNK),
                                       pl.ds(sl * 128, 128)],
                           rows_v[b], sems[b])

      def finish(i, b):
        @pl.when(i * NW + wid < nchunks)
        def _():
          base = (i * NW + wid) * CHUNK
          pltpu.make_async_copy(idx_hbm.at[pl.ds(base, CHUNK)], idx_v[b],
                                sems[b]).wait()
          pltpu.make_async_copy(rows_hbm.at[pl.ds(base, CHUNK),
                                            pl.ds(sl * 128, 128)],
                                rows_v[b], sems[b]).wait()
          pltpu.sync_copy(rows_v[b], acc.at[idx_v[b]], add=True)

      start(0, 0)

      def body(i2, carry):
        i = i2 * 2
        start(i + 1, 1)
        finish(i, 0)
        start(i + 2, 0)
        finish(i + 1, 1)
        return carry

      lax.fori_loop(0, (iters + 1) // 2, body, 0)
      plsc.subcore_barrier()
      pltpu.sync_copy(acc.at[pl.ds(s * rpn, rpn)],
                      out_hbm.at[c, sl].at[pl.ds(s * rpn, rpn)])
      plsc.subcore_barrier()

  return k(rows, idx, zeros)


def _mm_kernel(x_ref, w_ref, b_ref, o_ref):
  o_ref[...] = jnp.dot(x_ref[...], w_ref[...],
                       preferred_element_type=jnp.float32) + b_ref[...]


def _tc_matmul_bias(x, w, b, bn=400):
  n, kdim = x.shape
  m = w.shape[1]
  return pl.pallas_call(
      _mm_kernel,
      grid=(n // bn,),
      in_specs=[
          pl.BlockSpec((bn, kdim), lambda i: (i, 0)),
          pl.BlockSpec((kdim, m), lambda i: (0, 0)),
          pl.BlockSpec((1, m), lambda i: (0, 0)),
      ],
      out_specs=pl.BlockSpec((bn, m), lambda i: (i, 0)),
      out_shape=jax.ShapeDtypeStruct((n, m), jnp.float32),
  )(x, w, b.reshape(1, m))


def _node_prep_kernel(yd_ref, g_ref, bq_ref, bk_ref, bv_ref, wo_ref, bo_ref,
                      pool_ref, wq_ref, bq1_ref, wk_ref, bk1_ref,
                      wv_ref, bv1_ref, ws_ref, bs1_ref,
                      xin_ref, q_ref, k_ref, v_ref, sk_ref):
  dc0 = yd_ref[...]
  qf = jnp.dot(dc0, bq_ref[...], preferred_element_type=jnp.float32)
  kf = jnp.dot(dc0, bk_ref[...], preferred_element_type=jnp.float32)
  vf = jnp.dot(dc0, bv_ref[...], preferred_element_type=jnp.float32)
  s = jnp.dot(qf * kf, pool_ref[...],
              preferred_element_type=jnp.float32) / 12.0
  s = s - jnp.max(s, axis=-1, keepdims=True)
  es = jnp.exp(s)
  p = es / jnp.sum(es, axis=-1, keepdims=True)
  vw = vf * jnp.dot(p, pool_ref[...].T, preferred_element_type=jnp.float32)
  dc = jnp.dot(vw, wo_ref[...], preferred_element_type=jnp.float32) + bo_ref[...]
  g = g_ref[...]
  g = g / jnp.maximum(jnp.sqrt(jnp.sum(g * g, axis=-1, keepdims=True)), 1e-12)
  xin = jnp.concatenate([dc, g], axis=1)
  xin_ref[...] = xin
  q_ref[...] = jnp.dot(xin, wq_ref[...],
                       preferred_element_type=jnp.float32) + bq1_ref[...]
  k_ref[...] = jnp.dot(xin, wk_ref[...],
                       preferred_element_type=jnp.float32) + bk1_ref[...]
  v_ref[...] = jnp.dot(xin, wv_ref[...],
                       preferred_element_type=jnp.float32) + bv1_ref[...]
  sk_ref[...] = jnp.dot(xin, ws_ref[...],
                        preferred_element_type=jnp.float32) + bs1_ref[...]


def _tc_node_prep(yd, gemb, bq, bk, bv, wo, bo, pool,
                  wq, bq1, wk, bk1, wv, bv1, ws, bs1, bn=400):
  n = yd.shape[0]
  dp = wq.shape[1]
  full = lambda a: pl.BlockSpec(a.shape, lambda i: tuple(0 for _ in a.shape))
  row = lambda a: pl.BlockSpec((1, a.shape[0]), lambda i: (0, 0))
  return pl.pallas_call(
      _node_prep_kernel,
      grid=(n // bn,),
      in_specs=[
          pl.BlockSpec((bn, 256), lambda i: (i, 0)),
          pl.BlockSpec((bn, 128), lambda i: (i, 0)),
          full(bq), full(bk), full(bv), full(wo), row(bo),
          full(pool), full(wq), row(bq1), full(wk), row(bk1),
          full(wv), row(bv1), full(ws), row(bs1),
      ],
      out_specs=[
          pl.BlockSpec((bn, 272), lambda i: (i, 0)),
          pl.BlockSpec((bn, dp), lambda i: (i, 0)),
          pl.BlockSpec((bn, dp), lambda i: (i, 0)),
          pl.BlockSpec((bn, dp), lambda i: (i, 0)),
          pl.BlockSpec((bn, 272), lambda i: (i, 0)),
      ],
      out_shape=[
          jax.ShapeDtypeStruct((n, 272), jnp.float32),
          jax.ShapeDtypeStruct((n, dp), jnp.float32),
          jax.ShapeDtypeStruct((n, dp), jnp.float32),
          jax.ShapeDtypeStruct((n, dp), jnp.float32),
          jax.ShapeDtypeStruct((n, 272), jnp.float32),
      ],
  )(yd, gemb, bq, bk, bv, wo, bo.reshape(1, -1), pool,
    wq, bq1.reshape(1, -1), wk, bk1.reshape(1, -1),
    wv, bv1.reshape(1, -1), ws, bs1.reshape(1, -1))


def _edge_logits_kernel(qg_ref, kg_ref, oh_ref, ew_ref, hpool_ref,
                        l_ref, bmax_ref):
  eg = jnp.dot(oh_ref[...], ew_ref[...], preferred_element_type=jnp.float32)
  prod = qg_ref[...] * (kg_ref[...] + eg)
  l = jnp.dot(prod, hpool_ref[...],
              preferred_element_type=jnp.float32) / math.sqrt(float(HID))
  l_ref[...] = l
  bmax_ref[...] = jnp.max(l, axis=0, keepdims=True)[None]


def _tc_edge_logits(qg, kg, onehot, ew, hpool, be=2000):
  e, d = qg.shape
  h = hpool.shape[1]
  nb = e // be
  return pl.pallas_call(
      _edge_logits_kernel,
      grid=(nb,),
      in_specs=[
          pl.BlockSpec((be, d), lambda i: (i, 0)),
          pl.BlockSpec((be, d), lambda i: (i, 0)),
          pl.BlockSpec((be, 8), lambda i: (i, 0)),
          pl.BlockSpec((8, d), lambda i: (0, 0)),
          pl.BlockSpec((d, h), lambda i: (0, 0)),
      ],
      out_specs=[
          pl.BlockSpec((be, h), lambda i: (i, 0)),
          pl.BlockSpec((1, 1, h), lambda i: (i, 0, 0)),
      ],
      out_shape=[
          jax.ShapeDtypeStruct((e, h), jnp.float32),
          jax.ShapeDtypeStruct((nb, 1, h), jnp.float32),
      ],
  )(qg, kg, onehot, ew, hpool)


def _edge_weights_kernel(l_ref, gmax_ref, vg_ref, oh_ref, ew_ref, hexp_ref,
                         sel_ref, ext_ref):
  ex = jnp.exp(l_ref[...] - gmax_ref[...])
  eg = jnp.dot(oh_ref[...], ew_ref[...], preferred_element_type=jnp.float32)
  mult = jnp.dot(ex, hexp_ref[...], preferred_element_type=jnp.float32)
  ext_ref[...] = (vg_ref[...] + eg) * mult + jnp.dot(
      ex, sel_ref[...], preferred_element_type=jnp.float32)


def _tc_edge_weights(l, gmax, vg, onehot, ew, hexp, sel, be=2000):
  e, d = vg.shape
  h = l.shape[1]
  nb = e // be
  return pl.pallas_call(
      _edge_weights_kernel,
      grid=(nb,),
      in_specs=[
          pl.BlockSpec((be, h), lambda i: (i, 0)),
          pl.BlockSpec((1, h), lambda i: (0, 0)),
          pl.BlockSpec((be, d), lambda i: (i, 0)),
          pl.BlockSpec((be, 8), lambda i: (i, 0)),
          pl.BlockSpec((8, d), lambda i: (0, 0)),
          pl.BlockSpec((h, d), lambda i: (0, 0)),
          pl.BlockSpec((h, d), lambda i: (0, 0)),
      ],
      out_specs=pl.BlockSpec((be, d), lambda i: (i, 0)),
      out_shape=jax.ShapeDtypeStruct((e, d), jnp.float32),
  )(l, gmax, vg, onehot, ew, hexp, sel)


def _post_kernel(h, cat, xin_ref, num_ref, den_ref, sk_ref,
                 lng_ref, lnb_ref, wgate_ref, bgate_ref,
                 wq_ref, bq_ref, wk_ref, bk_ref, wv_ref, bv_ref,
                 ws_ref, bs_ref,
                 x_ref, q_ref, k_ref, v_ref, sk2_ref):
  num = num_ref[0] + num_ref[1]
  den = den_ref[0] + den_ref[1]
  xin = xin_ref[...]
  parts = []
  for hh in range(h):
    d = den[:, hh:hh + 1]
    inv = jnp.where(d > 0, 1.0 / jnp.where(d > 0, d, 1.0), 0.0)
    parts.append(num[:, hh, :] * inv)
  if cat:
    o = jnp.concatenate(parts, axis=1)
  else:
    o = parts[0]
    for pp in parts[1:]:
      o = o + pp
    o = o / float(h)
  x1 = o + sk_ref[...]
  mu = jnp.mean(x1, axis=-1, keepdims=True)
  var = jnp.mean((x1 - mu) ** 2, axis=-1, keepdims=True)
  x1 = (x1 - mu) * lax.rsqrt(var + 1e-5) * lng_ref[...] + lnb_ref[...]
  x1 = 0.5 * x1 * (1.0 + lax.erf(x1 / math.sqrt(2.0)))
  z = jnp.concatenate([xin, x1], axis=1)
  gate = jnp.dot(z, wgate_ref[...], preferred_element_type=jnp.float32)
  gate = jax.nn.sigmoid(gate + bgate_ref[...])[:, 0:1]
  x = gate * xin + (1.0 - gate) * x1
  x_ref[...] = x
  q_ref[...] = jnp.dot(x, wq_ref[...],
                       preferred_element_type=jnp.float32) + bq_ref[...]
  k_ref[...] = jnp.dot(x, wk_ref[...],
                       preferred_element_type=jnp.float32) + bk_ref[...]
  v_ref[...] = jnp.dot(x, wv_ref[...],
                       preferred_element_type=jnp.float32) + bv_ref[...]
  sk2_ref[...] = jnp.dot(x, ws_ref[...],
                         preferred_element_type=jnp.float32) + bs_ref[...]


def _tc_post(h, cat, xin, num, den, sk, lng, lnb, wgate, bgate,
             wq, bq, wk, bk, wv, bv, ws, bs, bn=400):
  n = xin.shape[0]
  dp = wq.shape[1]
  sp = ws.shape[1]
  gw = wgate.shape[1]
  full = lambda a: pl.BlockSpec(a.shape, lambda i: tuple(0 for _ in a.shape))
  row = lambda a: pl.BlockSpec((1, a.shape[0]), lambda i: (0, 0))
  return pl.pallas_call(
      functools.partial(_post_kernel, h, cat),
      grid=(n // bn,),
      in_specs=[
          pl.BlockSpec((bn, 272), lambda i: (i, 0)),
          pl.BlockSpec((2, bn, h, 272), lambda i: (0, i, 0, 0)),
          pl.BlockSpec((2, bn, 8), lambda i: (0, i, 0)),
          pl.BlockSpec((bn, 272), lambda i: (i, 0)),
          row(lng), row(lnb),
          full(wgate), row(bgate),
          full(wq), row(bq), full(wk), row(bk), full(wv), row(bv),
          full(ws), row(bs),
      ],
      out_specs=[
          pl.BlockSpec((bn, 272), lambda i: (i, 0)),
          pl.BlockSpec((bn, dp), lambda i: (i, 0)),
          pl.BlockSpec((bn, dp), lambda i: (i, 0)),
          pl.BlockSpec((bn, dp), lambda i: (i, 0)),
          pl.BlockSpec((bn, sp), lambda i: (i, 0)),
      ],
      out_shape=[
          jax.ShapeDtypeStruct((n, 272), jnp.float32),
          jax.ShapeDtypeStruct((n, dp), jnp.float32),
          jax.ShapeDtypeStruct((n, dp), jnp.float32),
          jax.ShapeDtypeStruct((n, dp), jnp.float32),
          jax.ShapeDtypeStruct((n, sp), jnp.float32),
      ],
  )(xin, num, den, sk, lng.reshape(1, -1), lnb.reshape(1, -1),
    wgate, bgate.reshape(1, -1), wq, bq.reshape(1, -1),
    wk, bk.reshape(1, -1), wv, bv.reshape(1, -1), ws, bs.reshape(1, -1))


def _padc(a, cols):
  return jnp.pad(a, ((0, 0), (0, cols - a.shape[1])))


def kernel(combined_embeddings, params, gene_node_indices, dna_node_indices,
           edge_index, edge_attr):
  pdict = params
  n = combined_embeddings.shape[0]

  wc = jnp.zeros((4096, 256), jnp.float32)
  wc = wc.at[0:768, 0:64].set(pdict['W1'].T)
  wc = wc.at[768:1536, 64:128].set(pdict['W2'].T)
  wc = wc.at[1536:4096, 128:144].set(pdict['W3'].T)
  bc = jnp.pad(jnp.concatenate([pdict['b1'], pdict['b2'], pdict['b3']]),
               (0, 112))
  eye4 = jnp.eye(4, dtype=jnp.float32)
  pad_rows = lambda a: jnp.pad(a, ((0, 256 - a.shape[0]), (0, 0)))
  bq = pad_rows(jnp.kron(eye4, pdict['mha_Wq'].T))
  bk = pad_rows(jnp.kron(eye4, pdict['mha_Wk'].T))
  bv = pad_rows(jnp.kron(eye4, pdict['mha_Wv'].T))
  pool4 = jnp.kron(eye4, jnp.ones((36, 1), jnp.float32))
  wo = pdict['mha_Wo'].T

  c1, c2 = pdict['conv1'], pdict['conv2']
  dp1, dp2 = 640, 384
  wq1 = _padc(c1['Wq'].T, dp1)
  wk1 = _padc(c1['Wk'].T, dp1)
  wv1 = _padc(c1['Wv'].T, dp1)
  bq1 = jnp.pad(c1['bq'], (0, dp1 - 544))
  bk1 = jnp.pad(c1['bk'], (0, dp1 - 544))
  bv1 = jnp.pad(c1['bv'], (0, dp1 - 544))
  ws1 = c1['Wskip'].T
  bs1 = c1['bskip']
  wq2 = _padc(c2['Wq'].T, dp2)
  wk2 = _padc(c2['Wk'].T, dp2)
  wv2 = _padc(c2['Wv'].T, dp2)
  bq2 = jnp.pad(c2['bq'], (0, dp2 - 272))
  bk2 = jnp.pad(c2['bk'], (0, dp2 - 272))
  bv2 = jnp.pad(c2['bv'], (0, dp2 - 272))
  ws2 = c2['Wskip'].T
  bs2 = c2['bskip']
  ew1 = _padc(pdict['pathway_emb'] @ c1['We'].T, dp1)
  ew2 = _padc(pdict['pathway_emb'] @ c2['We'].T, dp2)

  didx = dna_node_indices.astype(jnp.int32)
  gidx = jnp.clip(gene_node_indices, 0,
                  pdict['gene_emb'].shape[0] - 1).astype(jnp.int32)
  src = edge_index[0].astype(jnp.int32)
  dst = edge_index[1].astype(jnp.int32)
  pid = edge_attr[:, 0]
  pid = jnp.where(pid < 0, pdict['pathway_emb'].shape[0] - 1, pid)
  onehot = (pid[:, None] == jnp.arange(8)[None, :]).astype(jnp.float32)

  pad = NPAD - n
  didx_p = jnp.pad(didx, (0, pad))
  gidx_p = jnp.pad(gidx, (0, pad))
  zeros128 = jnp.zeros((NPAD, 128), jnp.float32)

  y = _tc_matmul_bias(combined_embeddings, wc, bc)
  yd = _sc_gather(y, didx_p)[:n]
  gemb = _sc_gather(pdict['gene_emb'], gidx_p)[:n]

  xin, q1, k1, v1, sk1 = _tc_node_prep(
      yd, gemb, bq, bk, bv, wo, pdict['mha_bo'], pool4,
      wq1, bq1, wk1, bk1, wv1, bv1, ws1, bs1)

  def edge_stage(q, k, v, ew, h):
    d = h * HID
    dpp = q.shape[1]
    nsl = dpp // 128
    hp = jnp.zeros((dpp, h), jnp.float32)
    for i in range(h):
      hp = hp.at[i * HID:(i + 1) * HID, i].set(1.0)
    sel = jnp.zeros((h, dpp), jnp.float32)
    for i in range(h):
      sel = sel.at[i, d + i].set(1.0)
    qg = _sc_gather(q, dst)
    kg = _sc_gather(k, src)
    vg = _sc_gather(v, src)
    l, bmax = _tc_edge_logits(qg, kg, onehot, ew, hp)
    gmax = jnp.max(bmax.reshape(-1, h), axis=0, keepdims=True)
    ext = _tc_edge_weights(l, gmax, vg, onehot, ew, hp.T, sel)
    parts = _sc_scatter_add(ext, dst, zeros128, nsl)
    full = parts.transpose(0, 2, 1, 3).reshape(2, NPAD, dpp)[:, :n]
    num = full[:, :, :d].reshape(2, n, h, HID)
    den = jnp.pad(full[:, :, d:d + h], ((0, 0), (0, 0), (0, 8 - h)))
    return num, den

  num1, den1 = edge_stage(q1, k1, v1, ew1, 2)
  gate1w = jnp.pad(pdict['bl1_W'].T, ((0, 0), (0, 7)))
  gate1b = jnp.pad(pdict['bl1_b'], (0, 7))
  x, q2, k2, v2, sk2 = _tc_post(
      2, False, xin, num1, den1, sk1, pdict['ln1_g'], pdict['ln1_b'],
      gate1w, gate1b, wq2, bq2, wk2, bk2, wv2, bv2, ws2, bs2)

  num2, den2 = edge_stage(q2, k2, v2, ew2, 1)
  gate2w = jnp.pad(pdict['bl2_W'].T, ((0, 0), (0, 7)))
  gate2b = jnp.pad(pdict['bl2_b'], (0, 7))
  zz = jnp.zeros((272, 128), jnp.float32)
  zb = jnp.zeros((128,), jnp.float32)
  out, _, _, _, _ = _tc_post(
      1, True, x, num2, den2, sk2, pdict['ln2_g'], pdict['ln2_b'],
      gate2w, gate2b, zz, zb, zz, zb, zz, zb, zz, zb)
  return out, pdict['pathway_emb']


# speedup vs baseline: 29.9772x; 1.0013x over previous
import functools
import math

import jax
import jax.numpy as jnp
from jax import lax
from jax.experimental import pallas as pl
from jax.experimental.pallas import tpu as pltpu
from jax.experimental.pallas import tpu_sc as plsc

HID = 272
NPAD = 10240
CHUNK = 128
NC, NS = 2, 16
NW = NC * NS


def _gchunk(d):
  for c, nb in ((128, 3), (64, 3), (32, 3), (16, 3), (8, 3)):
    if nb * c * d * 4 <= 500_000 and 160_000 % c == 0:
      return c, nb
  return 8, 2


def _sc_gather(table, idx):
  B = idx.shape[0]
  D = table.shape[1]
  chunk, nb = _gchunk(D)
  assert B % chunk == 0
  nchunks = B // chunk
  iters = (nchunks + NW - 1) // NW
  itersb = (iters + nb - 1) // nb
  mesh = plsc.VectorSubcoreMesh(core_axis_name="c", subcore_axis_name="s")

  @functools.partial(
      pl.kernel, mesh=mesh,
      out_type=jax.ShapeDtypeStruct((B, D), jnp.float32),
      scratch_types=(
          [pltpu.VMEM((chunk,), jnp.int32)] * nb
          + [pltpu.VMEM((chunk, D), jnp.float32)] * nb
          + [pltpu.SemaphoreType.DMA] * nb
      ),
  )
  def k(table_hbm, idx_hbm, out_hbm, *bufs):
    idx_v = bufs[0:nb]
    rows_v = bufs[nb:2 * nb]
    sems = bufs[2 * nb:3 * nb]
    wid = lax.axis_index("s") * NC + lax.axis_index("c")

    def start(i, b):
      @pl.when(i * NW + wid < nchunks)
      def _():
        base = (i * NW + wid) * chunk
        pltpu.sync_copy(idx_hbm.at[pl.ds(base, chunk)], idx_v[b])
        pltpu.async_copy(table_hbm.at[idx_v[b]], rows_v[b], sems[b])

    def finish(i, b):
      @pl.when(i * NW + wid < nchunks)
      def _():
        base = (i * NW + wid) * chunk
        pltpu.make_async_copy(table_hbm.at[idx_v[b]], rows_v[b],
                              sems[b]).wait()
        pltpu.sync_copy(rows_v[b], out_hbm.at[pl.ds(base, chunk)])

    for b in range(nb - 1):
      start(b, b)

    def body(ib, carry):
      i = ib * nb
      for b in range(nb):
        start(i + b + nb - 1, (b + nb - 1) % nb)
        finish(i + b, b)
      return carry

    lax.fori_loop(0, itersb, body, 0)

  return k(table, idx)


def _sc_scatter_add(rows, idx, zeros, nsl):
  E = rows.shape[0]
  nchunks = E // CHUNK
  iters = (nchunks + NW - 1) // NW
  rpn = NPAD // NS
  mesh = plsc.VectorSubcoreMesh(core_axis_name="c", subcore_axis_name="s")

  @functools.partial(
      pl.kernel, mesh=mesh,
      out_type=jax.ShapeDtypeStruct((NC, nsl, NPAD, 128), jnp.float32),
      scratch_types=[
          pltpu.VMEM((CHUNK,), jnp.int32),
          pltpu.VMEM((CHUNK,), jnp.int32),
          pltpu.VMEM((CHUNK, 128), jnp.float32),
          pltpu.VMEM((CHUNK, 128), jnp.float32),
          pltpu.VMEM_SHARED((NPAD, 128), jnp.float32),
          pltpu.SemaphoreType.DMA,
          pltpu.SemaphoreType.DMA,
      ],
  )
  def k(rows_hbm, idx_hbm, zeros_hbm, out_hbm, i0, i1, r0, r1, acc,
        s0, s1):
    c = lax.axis_index("c")
    s = lax.axis_index("s")
    wid = s * NC + c
    idx_v = (i0, i1)
    rows_v = (r0, r1)
    sems = (s0, s1)
    for sl in range(nsl):
      pltpu.sync_copy(zeros_hbm.at[pl.ds(s * rpn, rpn)],
                      acc.at[pl.ds(s * rpn, rpn)])
      plsc.subcore_barrier()

      def start(i, b):
        @pl.when(i * NW + wid < nchunks)
        def _():
          base = (i * NW + wid) * CHUNK
          pltpu.async_copy(idx_hbm.at[pl.ds(base, CHUNK)], idx_v[b], sems[b])
          pltpu.async_copy(rows_hbm.at[pl.ds(base, CHUNK),
                                       pl.ds(sl * 128, 128)],
                           rows_v[b], sems[b])

      def finish(i, b):
        @pl.when(i * NW + wid < nchunks)
        def _():
          base = (i * NW + wid) * CHUNK
          pltpu.make_async_copy(idx_hbm.at[pl.ds(base, CHUNK)], idx_v[b],
                                sems[b]).wait()
          pltpu.make_async_copy(rows_hbm.at[pl.ds(base, CHUNK),
                                            pl.ds(sl * 128, 128)],
                                rows_v[b], sems[b]).wait()
          pltpu.sync_copy(rows_v[b], acc.at[idx_v[b]], add=True)

      start(0, 0)

      def body(i2, carry):
        i = i2 * 2
        start(i + 1, 1)
        finish(i, 0)
        start(i + 2, 0)
        finish(i + 1, 1)
        return carry

      lax.fori_loop(0, (iters + 1) // 2, body, 0)
      plsc.subcore_barrier()
      pltpu.sync_copy(acc.at[pl.ds(s * rpn, rpn)],
                      out_hbm.at[c, sl].at[pl.ds(s * rpn, rpn)])
      plsc.subcore_barrier()

  return k(rows, idx, zeros)


def _mm_kernel(x_ref, w_ref, b_ref, o_ref):
  o_ref[...] = jnp.dot(x_ref[...], w_ref[...],
                       preferred_element_type=jnp.float32) + b_ref[...]


def _tc_matmul_bias(x, w, b, bn=400):
  n, kdim = x.shape
  m = w.shape[1]
  return pl.pallas_call(
      _mm_kernel,
      grid=(n // bn,),
      in_specs=[
          pl.BlockSpec((bn, kdim), lambda i: (i, 0)),
          pl.BlockSpec((kdim, m), lambda i: (0, 0)),
          pl.BlockSpec((1, m), lambda i: (0, 0)),
      ],
      out_specs=pl.BlockSpec((bn, m), lambda i: (i, 0)),
      out_shape=jax.ShapeDtypeStruct((n, m), jnp.float32),
  )(x, w, b.reshape(1, m))


def _node_prep_kernel(yd_ref, g_ref, bq_ref, bk_ref, bv_ref, wo_ref, bo_ref,
                      pool_ref, wq_ref, bq1_ref, wk_ref, bk1_ref,
                      wv_ref, bv1_ref, ws_ref, bs1_ref,
                      xin_ref, q_ref, k_ref, v_ref, sk_ref):
  dc0 = yd_ref[...]
  qf = jnp.dot(dc0, bq_ref[...], preferred_element_type=jnp.float32)
  kf = jnp.dot(dc0, bk_ref[...], preferred_element_type=jnp.float32)
  vf = jnp.dot(dc0, bv_ref[...], preferred_element_type=jnp.float32)
  s = jnp.dot(qf * kf, pool_ref[...],
              preferred_element_type=jnp.float32) / 12.0
  s = s - jnp.max(s, axis=-1, keepdims=True)
  es = jnp.exp(s)
  p = es / jnp.sum(es, axis=-1, keepdims=True)
  vw = vf * jnp.dot(p, pool_ref[...].T, preferred_element_type=jnp.float32)
  dc = jnp.dot(vw, wo_ref[...], preferred_element_type=jnp.float32) + bo_ref[...]
  g = g_ref[...]
  g = g / jnp.maximum(jnp.sqrt(jnp.sum(g * g, axis=-1, keepdims=True)), 1e-12)
  xin = jnp.concatenate([dc, g], axis=1)
  xin_ref[...] = xin
  q_ref[...] = jnp.dot(xin, wq_ref[...],
                       preferred_element_type=jnp.float32) + bq1_ref[...]
  k_ref[...] = jnp.dot(xin, wk_ref[...],
                       preferred_element_type=jnp.float32) + bk1_ref[...]
  v_ref[...] = jnp.dot(xin, wv_ref[...],
                       preferred_element_type=jnp.float32) + bv1_ref[...]
  sk_ref[...] = jnp.dot(xin, ws_ref[...],
                        preferred_element_type=jnp.float32) + bs1_ref[...]


def _tc_node_prep(yd, gemb, bq, bk, bv, wo, bo, pool,
                  wq, bq1, wk, bk1, wv, bv1, ws, bs1, bn=400):
  n = yd.shape[0]
  dp = wq.shape[1]
  full = lambda a: pl.BlockSpec(a.shape, lambda i: tuple(0 for _ in a.shape))
  row = lambda a: pl.BlockSpec((1, a.shape[0]), lambda i: (0, 0))
  return pl.pallas_call(
      _node_prep_kernel,
      grid=(n // bn,),
      in_specs=[
          pl.BlockSpec((bn, 256), lambda i: (i, 0)),
          pl.BlockSpec((bn, 128), lambda i: (i, 0)),
          full(bq), full(bk), full(bv), full(wo), row(bo),
          full(pool), full(wq), row(bq1), full(wk), row(bk1),
          full(wv), row(bv1), full(ws), row(bs1),
      ],
      out_specs=[
          pl.BlockSpec((bn, 272), lambda i: (i, 0)),
          pl.BlockSpec((bn, dp), lambda i: (i, 0)),
          pl.BlockSpec((bn, dp), lambda i: (i, 0)),
          pl.BlockSpec((bn, dp), lambda i: (i, 0)),
          pl.BlockSpec((bn, 272), lambda i: (i, 0)),
      ],
      out_shape=[
          jax.ShapeDtypeStruct((n, 272), jnp.float32),
          jax.ShapeDtypeStruct((n, dp), jnp.float32),
          jax.ShapeDtypeStruct((n, dp), jnp.float32),
          jax.ShapeDtypeStruct((n, dp), jnp.float32),
          jax.ShapeDtypeStruct((n, 272), jnp.float32),
      ],
  )(yd, gemb, bq, bk, bv, wo, bo.reshape(1, -1), pool,
    wq, bq1.reshape(1, -1), wk, bk1.reshape(1, -1),
    wv, bv1.reshape(1, -1), ws, bs1.reshape(1, -1))


def _edge_logits_kernel(qg_ref, kg_ref, oh_ref, ew_ref, hpool_ref,
                        l_ref, bmax_ref):
  eg = jnp.dot(oh_ref[...], ew_ref[...], preferred_element_type=jnp.float32)
  prod = qg_ref[...] * (kg_ref[...] + eg)
  l = jnp.dot(prod, hpool_ref[...],
              preferred_element_type=jnp.float32) / math.sqrt(float(HID))
  l_ref[...] = l
  bmax_ref[...] = jnp.max(l, axis=0, keepdims=True)[None]


def _tc_edge_logits(qg, kg, onehot, ew, hpool, be=2000):
  e, d = qg.shape
  h = hpool.shape[1]
  nb = e // be
  return pl.pallas_call(
      _edge_logits_kernel,
      grid=(nb,),
      in_specs=[
          pl.BlockSpec((be, d), lambda i: (i, 0)),
          pl.BlockSpec((be, d), lambda i: (i, 0)),
          pl.BlockSpec((be, 8), lambda i: (i, 0)),
          pl.BlockSpec((8, d), lambda i: (0, 0)),
          pl.BlockSpec((d, h), lambda i: (0, 0)),
      ],
      out_specs=[
          pl.BlockSpec((be, h), lambda i: (i, 0)),
          pl.BlockSpec((1, 1, h), lambda i: (i, 0, 0)),
      ],
      out_shape=[
          jax.ShapeDtypeStruct((e, h), jnp.float32),
          jax.ShapeDtypeStruct((nb, 1, h), jnp.float32),
      ],
  )(qg, kg, onehot, ew, hpool)


def _edge_weights_kernel(l_ref, gmax_ref, vg_ref, oh_ref, ew_ref, hexp_ref,
                         sel_ref, ext_ref):
  ex = jnp.exp(l_ref[...] - gmax_ref[...])
  eg = jnp.dot(oh_ref[...], ew_ref[...], preferred_element_type=jnp.float32)
  mult = jnp.dot(ex, hexp_ref[...], preferred_element_type=jnp.float32)
  ext_ref[...] = (vg_ref[...] + eg) * mult + jnp.dot(
      ex, sel_ref[...], preferred_element_type=jnp.float32)


def _tc_edge_weights(l, gmax, vg, onehot, ew, hexp, sel, be=2000):
  e, d = vg.shape
  h = l.shape[1]
  nb = e // be
  return pl.pallas_call(
      _edge_weights_kernel,
      grid=(nb,),
      in_specs=[
          pl.BlockSpec((be, h), lambda i: (i, 0)),
          pl.BlockSpec((1, h), lambda i: (0, 0)),
          pl.BlockSpec((be, d), lambda i: (i, 0)),
          pl.BlockSpec((be, 8), lambda i: (i, 0)),
          pl.BlockSpec((8, d), lambda i: (0, 0)),
          pl.BlockSpec((h, d), lambda i: (0, 0)),
          pl.BlockSpec((h, d), lambda i: (0, 0)),
      ],
      out_specs=pl.BlockSpec((be, d), lambda i: (i, 0)),
      out_shape=jax.ShapeDtypeStruct((e, d), jnp.float32),
  )(l, gmax, vg, onehot, ew, hexp, sel)


def _post_kernel(h, cat, xin_ref, num_ref, den_ref, sk_ref,
                 lng_ref, lnb_ref, wgate_ref, bgate_ref,
                 wq_ref, bq_ref, wk_ref, bk_ref, wv_ref, bv_ref,
                 ws_ref, bs_ref,
                 x_ref, q_ref, k_ref, v_ref, sk2_ref):
  num = num_ref[0] + num_ref[1]
  den = den_ref[0] + den_ref[1]
  xin = xin_ref[...]
  parts = []
  for hh in range(h):
    d = den[:, hh:hh + 1]
    inv = jnp.where(d > 0, 1.0 / jnp.where(d > 0, d, 1.0), 0.0)
    parts.append(num[:, hh, :] * inv)
  if cat:
    o = jnp.concatenate(parts, axis=1)
  else:
    o = parts[0]
    for pp in parts[1:]:
      o = o + pp
    o = o / float(h)
  x1 = o + sk_ref[...]
  mu = jnp.mean(x1, axis=-1, keepdims=True)
  var = jnp.mean((x1 - mu) ** 2, axis=-1, keepdims=True)
  x1 = (x1 - mu) * lax.rsqrt(var + 1e-5) * lng_ref[...] + lnb_ref[...]
  x1 = 0.5 * x1 * (1.0 + lax.erf(x1 / math.sqrt(2.0)))
  z = jnp.concatenate([xin, x1], axis=1)
  gate = jnp.dot(z, wgate_ref[...], preferred_element_type=jnp.float32)
  gate = jax.nn.sigmoid(gate + bgate_ref[...])[:, 0:1]
  x = gate * xin + (1.0 - gate) * x1
  x_ref[...] = x
  q_ref[...] = jnp.dot(x, wq_ref[...],
                       preferred_element_type=jnp.float32) + bq_ref[...]
  k_ref[...] = jnp.dot(x, wk_ref[...],
                       preferred_element_type=jnp.float32) + bk_ref[...]
  v_ref[...] = jnp.dot(x, wv_ref[...],
                       preferred_element_type=jnp.float32) + bv_ref[...]
  sk2_ref[...] = jnp.dot(x, ws_ref[...],
                         preferred_element_type=jnp.float32) + bs_ref[...]


def _tc_post(h, cat, xin, num, den, sk, lng, lnb, wgate, bgate,
             wq, bq, wk, bk, wv, bv, ws, bs, bn=400):
  n = xin.shape[0]
  dp = wq.shape[1]
  sp = ws.shape[1]
  gw = wgate.shape[1]
  full = lambda a: pl.BlockSpec(a.shape, lambda i: tuple(0 for _ in a.shape))
  row = lambda a: pl.BlockSpec((1, a.shape[0]), lambda i: (0, 0))
  return pl.pallas_call(
      functools.partial(_post_kernel, h, cat),
      grid=(n // bn,),
      in_specs=[
          pl.BlockSpec((bn, 272), lambda i: (i, 0)),
          pl.BlockSpec((2, bn, h, 272), lambda i: (0, i, 0, 0)),
          pl.BlockSpec((2, bn, 8), lambda i: (0, i, 0)),
          pl.BlockSpec((bn, 272), lambda i: (i, 0)),
          row(lng), row(lnb),
          full(wgate), row(bgate),
          full(wq), row(bq), full(wk), row(bk), full(wv), row(bv),
          full(ws), row(bs),
      ],
      out_specs=[
          pl.BlockSpec((bn, 272), lambda i: (i, 0)),
          pl.BlockSpec((bn, dp), lambda i: (i, 0)),
          pl.BlockSpec((bn, dp), lambda i: (i, 0)),
          pl.BlockSpec((bn, dp), lambda i: (i, 0)),
          pl.BlockSpec((bn, sp), lambda i: (i, 0)),
      ],
      out_shape=[
          jax.ShapeDtypeStruct((n, 272), jnp.float32),
          jax.ShapeDtypeStruct((n, dp), jnp.float32),
          jax.ShapeDtypeStruct((n, dp), jnp.float32),
          jax.ShapeDtypeStruct((n, dp), jnp.float32),
          jax.ShapeDtypeStruct((n, sp), jnp.float32),
      ],
  )(xin, num, den, sk, lng.reshape(1, -1), lnb.reshape(1, -1),
    wgate, bgate.reshape(1, -1), wq, bq.reshape(1, -1),
    wk, bk.reshape(1, -1), wv, bv.reshape(1, -1), ws, bs.reshape(1, -1))


def _padc(a, cols):
  return jnp.pad(a, ((0, 0), (0, cols - a.shape[1])))


def kernel(combined_embeddings, params, gene_node_indices, dna_node_indices,
           edge_index, edge_attr):
  pdict = params
  n = combined_embeddings.shape[0]

  wc = jnp.zeros((4096, 256), jnp.float32)
  wc = wc.at[0:768, 0:64].set(pdict['W1'].T)
  wc = wc.at[768:1536, 64:128].set(pdict['W2'].T)
  wc = wc.at[1536:4096, 128:144].set(pdict['W3'].T)
  bc = jnp.pad(jnp.concatenate([pdict['b1'], pdict['b2'], pdict['b3']]),
               (0, 112))
  eye4 = jnp.eye(4, dtype=jnp.float32)
  pad_rows = lambda a: jnp.pad(a, ((0, 256 - a.shape[0]), (0, 0)))
  bq = pad_rows(jnp.kron(eye4, pdict['mha_Wq'].T))
  bk = pad_rows(jnp.kron(eye4, pdict['mha_Wk'].T))
  bv = pad_rows(jnp.kron(eye4, pdict['mha_Wv'].T))
  pool4 = jnp.kron(eye4, jnp.ones((36, 1), jnp.float32))
  wo = pdict['mha_Wo'].T

  c1, c2 = pdict['conv1'], pdict['conv2']
  dp1, dp2 = 640, 384
  wq1 = _padc(c1['Wq'].T, dp1)
  wk1 = _padc(c1['Wk'].T, dp1)
  wv1 = _padc(c1['Wv'].T, dp1)
  bq1 = jnp.pad(c1['bq'], (0, dp1 - 544))
  bk1 = jnp.pad(c1['bk'], (0, dp1 - 544))
  bv1 = jnp.pad(c1['bv'], (0, dp1 - 544))
  ws1 = c1['Wskip'].T
  bs1 = c1['bskip']
  wq2 = _padc(c2['Wq'].T, dp2)
  wk2 = _padc(c2['Wk'].T, dp2)
  wv2 = _padc(c2['Wv'].T, dp2)
  bq2 = jnp.pad(c2['bq'], (0, dp2 - 272))
  bk2 = jnp.pad(c2['bk'], (0, dp2 - 272))
  bv2 = jnp.pad(c2['bv'], (0, dp2 - 272))
  ws2 = c2['Wskip'].T
  bs2 = c2['bskip']
  ew1 = _padc(pdict['pathway_emb'] @ c1['We'].T, dp1)
  ew2 = _padc(pdict['pathway_emb'] @ c2['We'].T, dp2)

  didx = dna_node_indices.astype(jnp.int32)
  gidx = jnp.clip(gene_node_indices, 0,
                  pdict['gene_emb'].shape[0] - 1).astype(jnp.int32)
  src = edge_index[0].astype(jnp.int32)
  dst = edge_index[1].astype(jnp.int32)
  pid = edge_attr[:, 0]
  pid = jnp.where(pid < 0, pdict['pathway_emb'].shape[0] - 1, pid)
  onehot = (pid[:, None] == jnp.arange(8)[None, :]).astype(jnp.float32)

  pad = NPAD - n
  didx_p = jnp.pad(didx, (0, pad))
  gidx_p = jnp.pad(gidx, (0, pad))
  zeros128 = jnp.zeros((NPAD, 128), jnp.float32)

  y = _tc_matmul_bias(combined_embeddings, wc, bc)
  yd = _sc_gather(y, didx_p)[:n]
  gemb = _sc_gather(pdict['gene_emb'], gidx_p)[:n]

  xin, q1, k1, v1, sk1 = _tc_node_prep(
      yd, gemb, bq, bk, bv, wo, pdict['mha_bo'], pool4,
      wq1, bq1, wk1, bk1, wv1, bv1, ws1, bs1)

  def edge_stage(q, k, v, ew, h):
    d = h * HID
    dpp = q.shape[1]
    nsl = dpp // 128
    hp = jnp.zeros((dpp, h), jnp.float32)
    for i in range(h):
      hp = hp.at[i * HID:(i + 1) * HID, i].set(1.0)
    sel = jnp.zeros((h, dpp), jnp.float32)
    for i in range(h):
      sel = sel.at[i, d + i].set(1.0)
    qg = _sc_gather(q, dst)
    kg = _sc_gather(k, src)
    vg = _sc_gather(v, src)
    l, bmax = _tc_edge_logits(qg, kg, onehot, ew, hp)
    gmax = jnp.max(bmax.reshape(-1, h), axis=0, keepdims=True)
    ext = _tc_edge_weights(l, gmax, vg, onehot, ew, hp.T, sel)
    parts = _sc_scatter_add(ext, dst, zeros128, nsl)
    full = parts.transpose(0, 2, 1, 3).reshape(2, NPAD, dpp)[:, :n]
    num = full[:, :, :d].reshape(2, n, h, HID)
    den = jnp.pad(full[:, :, d:d + h], ((0, 0), (0, 0), (0, 8 - h)))
    return num, den

  num1, den1 = edge_stage(q1, k1, v1, ew1, 2)
  gate1w = jnp.pad(pdict['bl1_W'].T, ((0, 0), (0, 7)))
  gate1b = jnp.pad(pdict['bl1_b'], (0, 7))
  x, q2, k2, v2, sk2 = _tc_post(
      2, False, xin, num1, den1, sk1, pdict['ln1_g'], pdict['ln1_b'],
      gate1w, gate1b, wq2, bq2, wk2, bk2, wv2, bv2, ws2, bs2)

  num2, den2 = edge_stage(q2, k2, v2, ew2, 1)
  gate2w = jnp.pad(pdict['bl2_W'].T, ((0, 0), (0, 7)))
  gate2b = jnp.pad(pdict['bl2_b'], (0, 7))
  zz = jnp.zeros((272, 128), jnp.float32)
  zb = jnp.zeros((128,), jnp.float32)
  out, _, _, _, _ = _tc_post(
      1, True, x, num2, den2, sk2, pdict['ln2_g'], pdict['ln2_b'],
      gate2w, gate2b, zz, zb, zz, zb, zz, zb, zz, zb)
  return out, pdict['pathway_emb']
